# baseline (speedup 1.0000x reference)
import jax
import jax.numpy as jnp
from jax.experimental import pallas as pl
from jax.experimental.pallas import tpu as pltpu

_E = 16
_K = 2
_LBW = 0.01
_B, _S, _D = 4, 2048, 2048

_S_CHUNK = 512
_S_CHUNKS = _S // _S_CHUNK
_D_CHUNK = 512
_D_CHUNKS = _D // _D_CHUNK


def _gate_kernel(x_ref, w1_ref, b1_ref, w2_ref, b2_ref,
                 out_ref, acc_ref, hid_ref):
    c = pl.program_id(0)
    s = pl.program_id(1)

    @pl.when(s == 0)
    def _init():
        acc_ref[...] = jnp.zeros_like(acc_ref)

    acc_ref[...] += jnp.sum(x_ref[...], axis=1)

    @pl.when(s == _S_CHUNKS - 1)
    def _mm1():
        partial = jnp.dot(acc_ref[...] * (1.0 / _S), w1_ref[...],
                          preferred_element_type=jnp.float32)

        @pl.when(c == 0)
        def _set():
            hid_ref[...] = partial

        @pl.when(c > 0)
        def _add():
            hid_ref[...] += partial

    @pl.when((c == _D_CHUNKS - 1) & (s == _S_CHUNKS - 1))
    def _tail():
        h = hid_ref[...] + b1_ref[...]
        h = h * jax.nn.sigmoid(h)
        logits = jnp.dot(h, w2_ref[...],
                         preferred_element_type=jnp.float32) + b2_ref[...]
        iota = jax.lax.broadcasted_iota(jnp.int32, (_B, _E), 1)
        m1 = jnp.max(logits, axis=1, keepdims=True)
        i1 = jnp.min(jnp.where(logits == m1, iota, _E), axis=1, keepdims=True)
        masked = jnp.where(iota == i1, -jnp.inf, logits)
        m2 = jnp.max(masked, axis=1, keepdims=True)
        i2 = jnp.min(jnp.where(masked == m2, iota, _E), axis=1, keepdims=True)
        e2 = jnp.exp(m2 - m1)
        denom = 1.0 + e2
        k_iota = jax.lax.broadcasted_iota(jnp.int32, (_B, _K), 1)
        wpair = jnp.where(k_iota == 0, 1.0 / denom, e2 / denom)
        ipair = jnp.where(k_iota == 0, i1, i2).astype(jnp.float32)
        p = jnp.exp(logits - m1)
        p = p / jnp.sum(p, axis=1, keepdims=True)
        mean_gate_prob = jnp.mean(p, axis=0, keepdims=True)
        usage = jnp.sum((iota == i1).astype(jnp.float32)
                        + (iota == i2).astype(jnp.float32),
                        axis=0, keepdims=True)
        mean_usage = usage * (1.0 / (_B * _K))
        loss = _E * jnp.sum(mean_gate_prob * mean_usage)
        out_ref[0:_B, 0:_K] = wpair
        out_ref[_B:2 * _B, 0:_K] = ipair
        out_ref[0:1, 4:5] = jnp.full((1, 1), _LBW, jnp.float32) * loss


def kernel(x, W1, b1, W2, b2):
    b1r = b1.reshape(1, _D)
    b2r = b2.reshape(1, _E)
    packed = pl.pallas_call(
        _gate_kernel,
        grid=(_D_CHUNKS, _S_CHUNKS),
        in_specs=[
            pl.BlockSpec((_B, _S_CHUNK, _D_CHUNK), lambda c, s: (0, s, c)),
            pl.BlockSpec((_D_CHUNK, _D), lambda c, s: (c, 0)),
            pl.BlockSpec((1, _D), lambda c, s: (0, 0)),
            pl.BlockSpec((_D, _E), lambda c, s: (0, 0)),
            pl.BlockSpec((1, _E), lambda c, s: (0, 0)),
        ],
        out_specs=pl.BlockSpec((8, 128), lambda c, s: (0, 0)),
        out_shape=jax.ShapeDtypeStruct((8, 128), jnp.float32),
        scratch_shapes=[
            pltpu.VMEM((_B, _D_CHUNK), jnp.float32),
            pltpu.VMEM((_B, _D), jnp.float32),
        ],
    )(x, W1, b1r, W2, b2r)
    w = packed[0:_B, 0:_K]
    idx = packed[_B:2 * _B, 0:_K].astype(jnp.int32)
    loss = packed[0, 4]
    return (w, idx, loss)


# speedup vs baseline: 1.0719x; 1.0719x over previous
import jax
import jax.numpy as jnp
from jax.experimental import pallas as pl
from jax.experimental.pallas import tpu as pltpu

_E = 16
_K = 2
_LBW = 0.01
_B, _S, _D = 4, 2048, 2048

_S_CHUNK = 512
_S_CHUNKS = _S // _S_CHUNK
_D_CHUNK = 512
_D_CHUNKS = _D // _D_CHUNK


def _gate_kernel(x_ref, w1_ref, b1_ref, w2_ref, b2_ref,
                 out_ref, acc_ref, hid_ref):
    c = pl.program_id(0)
    s = pl.program_id(1)

    @pl.when(s == 0)
    def _init():
        acc_ref[...] = jnp.zeros_like(acc_ref)

    acc_ref[...] += jnp.sum(x_ref[...], axis=1)

    @pl.when(s == _S_CHUNKS - 1)
    def _mm1():
        partial = jnp.dot(acc_ref[...] * (1.0 / _S), w1_ref[...],
                          preferred_element_type=jnp.float32)

        @pl.when(c == 0)
        def _set():
            hid_ref[...] = partial

        @pl.when(c > 0)
        def _add():
            hid_ref[...] += partial

    @pl.when((c == _D_CHUNKS - 1) & (s == _S_CHUNKS - 1))
    def _tail():
        h = hid_ref[...] + b1_ref[...]
        h = h * jax.nn.sigmoid(h)
        logits = jax.lax.dot_general(
            h, w2_ref[...], (((1,), (1,)), ((), ())),
            preferred_element_type=jnp.float32) + b2_ref[...]
        iota = jax.lax.broadcasted_iota(jnp.int32, (_B, _E), 1)
        m1 = jnp.max(logits, axis=1, keepdims=True)
        i1 = jnp.min(jnp.where(logits == m1, iota, _E), axis=1, keepdims=True)
        masked = jnp.where(iota == i1, -jnp.inf, logits)
        m2 = jnp.max(masked, axis=1, keepdims=True)
        i2 = jnp.min(jnp.where(masked == m2, iota, _E), axis=1, keepdims=True)
        e2 = jnp.exp(m2 - m1)
        denom = 1.0 + e2
        k_iota = jax.lax.broadcasted_iota(jnp.int32, (_B, _K), 1)
        wpair = jnp.where(k_iota == 0, 1.0 / denom, e2 / denom)
        ipair = jnp.where(k_iota == 0, i1, i2).astype(jnp.float32)
        p = jnp.exp(logits - m1)
        p = p / jnp.sum(p, axis=1, keepdims=True)
        mean_gate_prob = jnp.mean(p, axis=0, keepdims=True)
        usage = jnp.sum((iota == i1).astype(jnp.float32)
                        + (iota == i2).astype(jnp.float32),
                        axis=0, keepdims=True)
        mean_usage = usage * (1.0 / (_B * _K))
        loss = _E * jnp.sum(mean_gate_prob * mean_usage)
        out_ref[0:_B, 0:_K] = wpair
        out_ref[_B:2 * _B, 0:_K] = ipair
        out_ref[0:1, 4:5] = jnp.full((1, 1), _LBW, jnp.float32) * loss


def kernel(x, W1, b1, W2, b2):
    b1r = b1.reshape(1, _D)
    b2r = b2.reshape(1, _E)
    w2r = W2.T
    packed = pl.pallas_call(
        _gate_kernel,
        grid=(_D_CHUNKS, _S_CHUNKS),
        in_specs=[
            pl.BlockSpec((_B, _S_CHUNK, _D_CHUNK), lambda c, s: (0, s, c)),
            pl.BlockSpec((_D_CHUNK, _D), lambda c, s: (c, 0)),
            pl.BlockSpec((1, _D), lambda c, s: (0, 0)),
            pl.BlockSpec((_E, _D), lambda c, s: (0, 0)),
            pl.BlockSpec((1, _E), lambda c, s: (0, 0)),
        ],
        out_specs=pl.BlockSpec((8, 128), lambda c, s: (0, 0)),
        out_shape=jax.ShapeDtypeStruct((8, 128), jnp.float32),
        scratch_shapes=[
            pltpu.VMEM((_B, _D_CHUNK), jnp.float32),
            pltpu.VMEM((_B, _D), jnp.float32),
        ],
    )(x, W1, b1r, w2r, b2r)
    w = packed[0:_B, 0:_K]
    idx = packed[_B:2 * _B, 0:_K].astype(jnp.int32)
    loss = packed[0, 4]
    return (w, idx, loss)
